# baseline (speedup 1.0000x reference)
import functools

import jax
import jax.numpy as jnp
from jax import lax
from jax.experimental import pallas as pl
from jax.experimental.pallas import tpu as pltpu
from jax.experimental.pallas import tpu_sc as plsc

B = 16384
NUM_NUMERIC = 13
NUM_CAT = 26
VOCAB = 1000
EMB_DIM = 4
ROW_OUT = NUM_CAT * EMB_DIM + NUM_NUMERIC
ROW_PAD = 120
OUT_WORDS = (ROW_PAD // 8) * B * 8

HALF = B // 2
QUARTER = B // 4
UNROLL = 16


def _ilv(pos):
  return pos + (pos >> 7) * 896


def _sc_embed(x_cols, tbl_f):
  mesh = plsc.VectorSubcoreMesh(core_axis_name="c", subcore_axis_name="s")

  @functools.partial(
      pl.kernel,
      out_type=jax.ShapeDtypeStruct((OUT_WORDS,), jnp.float32),
      mesh=mesh,
      scratch_types=[
          pltpu.VMEM((8 * HALF,), jnp.float32),
          pltpu.VMEM((HALF,), jnp.float32),
          pltpu.VMEM((HALF,), jnp.float32),
          pltpu.VMEM((2 * EMB_DIM * VOCAB,), jnp.float32),
          pltpu.SemaphoreType.DMA,
          pltpu.SemaphoreType.DMA,
      ],
      compiler_params=pltpu.CompilerParams(needs_layout_passes=False),
  )
  def k(x_hbm, tbl_hbm, out_hbm, emb_v, ids0_v, ids1_v, tbl_v, sem_in,
        sem_out):
    wid = lax.axis_index("s") * 2 + lax.axis_index("c")
    role = wid // 2
    h = wid % 2

    @pl.when(wid < NUM_CAT)
    def _field_pair():
      f0 = role * 2
      d_tbl = pltpu.async_copy(
          tbl_hbm.at[pl.ds(f0 * (EMB_DIM * VOCAB), 2 * EMB_DIM * VOCAB)],
          tbl_v, sem_in)
      d_i0 = pltpu.async_copy(
          x_hbm.at[pl.ds((NUM_NUMERIC + f0) * B + h * HALF, HALF)], ids0_v,
          sem_in)
      d_i1 = pltpu.async_copy(
          x_hbm.at[pl.ds((NUM_NUMERIC + f0 + 1) * B + h * HALF, HALF)],
          ids1_v, sem_in)
      d_tbl.wait()
      d_i0.wait()
      d_i1.wait()
      out_base = (role * 2 + h) * 8 * HALF
      outs = []
      for q in range(2):

        @plsc.parallel_loop(q * (HALF // 2), (q + 1) * (HALF // 2), step=16,
                            unroll=UNROLL)
        def _body(pos):
          lb = _ilv(pos)
          ids0 = ids0_v[pl.ds(pos, 16)].astype(jnp.int32)
          for d in range(EMB_DIM):
            v = plsc.load_gather(tbl_v, [ids0 + d * VOCAB])
            emb_v[pl.ds(lb + d * 128, 16)] = v
          ids1 = ids1_v[pl.ds(pos, 16)].astype(jnp.int32)
          for d in range(EMB_DIM):
            v = plsc.load_gather(tbl_v, [ids1 + (EMB_DIM + d) * VOCAB])
            emb_v[pl.ds(lb + (EMB_DIM + d) * 128, 16)] = v

        outs.append(pltpu.async_copy(
            emb_v.at[pl.ds(q * 4 * HALF, 4 * HALF)],
            out_hbm.at[pl.ds(out_base + q * 4 * HALF, 4 * HALF)], sem_out))
      for d_o in outs:
        d_o.wait()

    def _numeric(base_col, ncols, rbase, nrows, out_off):
      bufs = (ids0_v, ids1_v)
      pend = pltpu.async_copy(
          x_hbm.at[pl.ds(base_col * B + rbase, nrows)],
          bufs[0].at[pl.ds(0, nrows)], sem_in)
      for j in range(ncols):
        nxt = None
        if j + 1 < ncols:
          nxt = pltpu.async_copy(
              x_hbm.at[pl.ds((base_col + j + 1) * B + rbase, nrows)],
              bufs[(j + 1) % 2].at[pl.ds(0, nrows)], sem_in)
        pend.wait()
        pend = nxt
        buf = bufs[j % 2]

        @plsc.parallel_loop(0, nrows, step=16, unroll=UNROLL)
        def _copy(pos, j=j, buf=buf):
          emb_v[pl.ds(_ilv(pos) + j * 128, 16)] = buf[pl.ds(pos, 16)]

      pltpu.sync_copy(emb_v.at[pl.ds(0, 8 * nrows)],
                      out_hbm.at[pl.ds(out_off, 8 * nrows)])

    for q in range(4):

      @pl.when(wid == NUM_CAT + q)
      def _numeric_a(q=q):
        _numeric(0, 8, q * QUARTER, QUARTER,
                 13 * (8 * B) + q * 8 * QUARTER)

    for h2 in range(2):

      @pl.when(wid == NUM_CAT + 4 + h2)
      def _numeric_b(h2=h2):
        _numeric(8, 5, h2 * HALF, HALF, 14 * (8 * B) + h2 * 8 * HALF)

  return k(x_cols, tbl_f)


def kernel(x, tables):
  x_cols = x.T.reshape(-1)
  tbl_f = tables.transpose(0, 2, 1).reshape(-1)
  out_phys = _sc_embed(x_cols, tbl_f)
  out = (out_phys.reshape(ROW_PAD // 8, B // 128, 8, 128)
         .transpose(1, 3, 0, 2)
         .reshape(B, ROW_PAD)[:, :ROW_OUT])
  return out


# speedup vs baseline: 1.0414x; 1.0414x over previous
import functools

import jax
import jax.numpy as jnp
from jax import lax
from jax.experimental import pallas as pl
from jax.experimental.pallas import tpu as pltpu
from jax.experimental.pallas import tpu_sc as plsc

B = 16384
NUM_NUMERIC = 13
NUM_CAT = 26
VOCAB = 1000
EMB_DIM = 4
ROW_OUT = NUM_CAT * EMB_DIM + NUM_NUMERIC
ROW_PAD = 120
OUT_WORDS = (ROW_PAD // 8) * B * 8

HALF = B // 2
QUARTER = B // 4
UNROLL = 8


def _ilv(pos):
  return pos + (pos >> 7) * 896


def _sc_embed(x_cols, tbl_f):
  mesh = plsc.VectorSubcoreMesh(core_axis_name="c", subcore_axis_name="s")

  @functools.partial(
      pl.kernel,
      out_type=jax.ShapeDtypeStruct((OUT_WORDS,), jnp.float32),
      mesh=mesh,
      scratch_types=[
          pltpu.VMEM((8 * HALF,), jnp.float32),
          pltpu.VMEM((HALF,), jnp.float32),
          pltpu.VMEM((HALF,), jnp.float32),
          pltpu.VMEM((2 * EMB_DIM * VOCAB,), jnp.float32),
          pltpu.SemaphoreType.DMA,
          pltpu.SemaphoreType.DMA,
          pltpu.SemaphoreType.DMA,
      ],
      compiler_params=pltpu.CompilerParams(needs_layout_passes=False),
  )
  def k(x_hbm, tbl_hbm, out_hbm, emb_v, ids0_v, ids1_v, tbl_v, sem_in,
        sem_in2, sem_out):
    wid = lax.axis_index("s") * 2 + lax.axis_index("c")
    role = wid // 2
    h = wid % 2

    @pl.when(wid < NUM_CAT)
    def _field_pair():
      f0 = role * 2
      quarter = HALF // 2
      first, second = [], []
      first.append(pltpu.async_copy(
          tbl_hbm.at[pl.ds(f0 * (EMB_DIM * VOCAB), 2 * EMB_DIM * VOCAB)],
          tbl_v, sem_in))
      for c, ids_v in ((f0, ids0_v), (f0 + 1, ids1_v)):
        first.append(pltpu.async_copy(
            x_hbm.at[pl.ds((NUM_NUMERIC + c) * B + h * HALF, quarter)],
            ids_v.at[pl.ds(0, quarter)], sem_in))
        second.append(pltpu.async_copy(
            x_hbm.at[pl.ds((NUM_NUMERIC + c) * B + h * HALF + quarter,
                           quarter)],
            ids_v.at[pl.ds(quarter, quarter)], sem_in2))
      out_base = (role * 2 + h) * 8 * HALF
      outs = []
      for q in range(2):
        for d_i in (first if q == 0 else second):
          d_i.wait()

        @plsc.parallel_loop(q * quarter, (q + 1) * quarter, step=16,
                            unroll=UNROLL)
        def _body(pos):
          lb = _ilv(pos)
          ids0 = ids0_v[pl.ds(pos, 16)].astype(jnp.int32)
          for d in range(EMB_DIM):
            v = plsc.load_gather(tbl_v, [ids0 + d * VOCAB])
            emb_v[pl.ds(lb + d * 128, 16)] = v
          ids1 = ids1_v[pl.ds(pos, 16)].astype(jnp.int32)
          for d in range(EMB_DIM):
            v = plsc.load_gather(tbl_v, [ids1 + (EMB_DIM + d) * VOCAB])
            emb_v[pl.ds(lb + (EMB_DIM + d) * 128, 16)] = v

        outs.append(pltpu.async_copy(
            emb_v.at[pl.ds(q * 4 * HALF, 4 * HALF)],
            out_hbm.at[pl.ds(out_base + q * 4 * HALF, 4 * HALF)], sem_out))
      for d_o in outs:
        d_o.wait()

    def _numeric(base_col, ncols, rbase, nrows, out_off):
      bufs = (ids0_v, ids1_v)
      pend = pltpu.async_copy(
          x_hbm.at[pl.ds(base_col * B + rbase, nrows)],
          bufs[0].at[pl.ds(0, nrows)], sem_in)
      for j in range(ncols):
        nxt = None
        if j + 1 < ncols:
          nxt = pltpu.async_copy(
              x_hbm.at[pl.ds((base_col + j + 1) * B + rbase, nrows)],
              bufs[(j + 1) % 2].at[pl.ds(0, nrows)], sem_in)
        pend.wait()
        pend = nxt
        buf = bufs[j % 2]

        @plsc.parallel_loop(0, nrows, step=16, unroll=UNROLL)
        def _copy(pos, j=j, buf=buf):
          emb_v[pl.ds(_ilv(pos) + j * 128, 16)] = buf[pl.ds(pos, 16)]

      pltpu.sync_copy(emb_v.at[pl.ds(0, 8 * nrows)],
                      out_hbm.at[pl.ds(out_off, 8 * nrows)])

    for q in range(4):

      @pl.when(wid == NUM_CAT + q)
      def _numeric_a(q=q):
        _numeric(0, 8, q * QUARTER, QUARTER,
                 13 * (8 * B) + q * 8 * QUARTER)

    for h2 in range(2):

      @pl.when(wid == NUM_CAT + 4 + h2)
      def _numeric_b(h2=h2):
        _numeric(8, 5, h2 * HALF, HALF, 14 * (8 * B) + h2 * 8 * HALF)

  return k(x_cols, tbl_f)


def kernel(x, tables):
  x_cols = x.T.reshape(-1)
  tbl_f = tables.transpose(0, 2, 1).reshape(-1)
  out_phys = _sc_embed(x_cols, tbl_f)
  out = (out_phys.reshape(ROW_PAD // 8, B // 128, 8, 128)
         .transpose(1, 3, 0, 2)
         .reshape(B, ROW_PAD)[:, :ROW_OUT])
  return out


# speedup vs baseline: 1.1200x; 1.0755x over previous
import functools

import jax
import jax.numpy as jnp
from jax import lax
from jax.experimental import pallas as pl
from jax.experimental.pallas import tpu as pltpu
from jax.experimental.pallas import tpu_sc as plsc

B = 16384
NUM_NUMERIC = 13
NUM_CAT = 26
VOCAB = 1000
EMB_DIM = 4
ROW_OUT = NUM_CAT * EMB_DIM + NUM_NUMERIC
ROW_PAD = 120
OUT_WORDS = (ROW_PAD // 8) * B * 8

HALF = B // 2
QUARTER = B // 4
UNROLL = 8


def _ilv(pos):
  return pos + (pos >> 7) * 896


def _sc_embed(x_cols, tbl_f):
  mesh = plsc.VectorSubcoreMesh(core_axis_name="c", subcore_axis_name="s")

  @functools.partial(
      pl.kernel,
      out_type=jax.ShapeDtypeStruct((OUT_WORDS,), jnp.float32),
      mesh=mesh,
      scratch_types=[
          pltpu.VMEM((8 * HALF,), jnp.float32),
          pltpu.VMEM((HALF,), jnp.float32),
          pltpu.VMEM((HALF,), jnp.float32),
          pltpu.VMEM((2 * EMB_DIM * VOCAB,), jnp.float32),
          pltpu.SemaphoreType.DMA,
          pltpu.SemaphoreType.DMA,
          pltpu.SemaphoreType.DMA,
      ],
      compiler_params=pltpu.CompilerParams(needs_layout_passes=False),
  )
  def k(x_hbm, tbl_hbm, out_hbm, emb_v, ids0_v, ids1_v, tbl_v, sem_in,
        sem_in2, sem_out):
    wid = lax.axis_index("s") * 2 + lax.axis_index("c")
    role = wid // 2
    h = wid % 2

    @pl.when(wid < NUM_CAT)
    def _field_pair():
      f0 = role * 2
      quarter = HALF // 2
      first, second = [], []
      first.append(pltpu.async_copy(
          tbl_hbm.at[pl.ds(f0 * (EMB_DIM * VOCAB), 2 * EMB_DIM * VOCAB)],
          tbl_v, sem_in))
      for c, ids_v in ((f0, ids0_v), (f0 + 1, ids1_v)):
        first.append(pltpu.async_copy(
            x_hbm.at[NUM_NUMERIC + c, pl.ds(h * HALF, quarter)],
            ids_v.at[pl.ds(0, quarter)], sem_in))
        second.append(pltpu.async_copy(
            x_hbm.at[NUM_NUMERIC + c, pl.ds(h * HALF + quarter, quarter)],
            ids_v.at[pl.ds(quarter, quarter)], sem_in2))
      out_base = (role * 2 + h) * 8 * HALF
      outs = []
      for q in range(2):
        for d_i in (first if q == 0 else second):
          d_i.wait()

        @plsc.parallel_loop(q * quarter, (q + 1) * quarter, step=16,
                            unroll=UNROLL)
        def _body(pos):
          lb = _ilv(pos)
          ids0 = ids0_v[pl.ds(pos, 16)].astype(jnp.int32)
          for d in range(EMB_DIM):
            v = plsc.load_gather(tbl_v, [ids0 + d * VOCAB])
            emb_v[pl.ds(lb + d * 128, 16)] = v
          ids1 = ids1_v[pl.ds(pos, 16)].astype(jnp.int32)
          for d in range(EMB_DIM):
            v = plsc.load_gather(tbl_v, [ids1 + (EMB_DIM + d) * VOCAB])
            emb_v[pl.ds(lb + (EMB_DIM + d) * 128, 16)] = v

        outs.append(pltpu.async_copy(
            emb_v.at[pl.ds(q * 4 * HALF, 4 * HALF)],
            out_hbm.at[pl.ds(out_base + q * 4 * HALF, 4 * HALF)], sem_out))
      for d_o in outs:
        d_o.wait()

    def _numeric(base_col, ncols, rbase, nrows, out_off):
      bufs = (ids0_v, ids1_v)
      pend = pltpu.async_copy(
          x_hbm.at[base_col, pl.ds(rbase, nrows)],
          bufs[0].at[pl.ds(0, nrows)], sem_in)
      for j in range(ncols):
        nxt = None
        if j + 1 < ncols:
          nxt = pltpu.async_copy(
              x_hbm.at[base_col + j + 1, pl.ds(rbase, nrows)],
              bufs[(j + 1) % 2].at[pl.ds(0, nrows)], sem_in)
        pend.wait()
        pend = nxt
        buf = bufs[j % 2]

        @plsc.parallel_loop(0, nrows, step=16, unroll=UNROLL)
        def _copy(pos, j=j, buf=buf):
          emb_v[pl.ds(_ilv(pos) + j * 128, 16)] = buf[pl.ds(pos, 16)]

      pltpu.sync_copy(emb_v.at[pl.ds(0, 8 * nrows)],
                      out_hbm.at[pl.ds(out_off, 8 * nrows)])

    for q in range(4):

      @pl.when(wid == NUM_CAT + q)
      def _numeric_a(q=q):
        _numeric(0, 8, q * QUARTER, QUARTER,
                 13 * (8 * B) + q * 8 * QUARTER)

    for h2 in range(2):

      @pl.when(wid == NUM_CAT + 4 + h2)
      def _numeric_b(h2=h2):
        _numeric(8, 5, h2 * HALF, HALF, 14 * (8 * B) + h2 * 8 * HALF)

  return k(x_cols, tbl_f)


def kernel(x, tables):
  x_cols = x.T
  tbl_f = tables.transpose(0, 2, 1).reshape(-1)
  out_phys = _sc_embed(x_cols, tbl_f)
  out = (out_phys.reshape(ROW_PAD // 8, B // 128, 8, 128)
         .transpose(1, 3, 0, 2)
         .reshape(B, ROW_PAD)[:, :ROW_OUT])
  return out


# speedup vs baseline: 1.1216x; 1.0014x over previous
import functools

import jax
import jax.numpy as jnp
from jax import lax
from jax.experimental import pallas as pl
from jax.experimental.pallas import tpu as pltpu
from jax.experimental.pallas import tpu_sc as plsc

B = 16384
NUM_NUMERIC = 13
NUM_CAT = 26
VOCAB = 1000
EMB_DIM = 4
ROW_OUT = NUM_CAT * EMB_DIM + NUM_NUMERIC
ROW_PAD = 120
OUT_WORDS = (ROW_PAD // 8) * B * 8

HALF = B // 2
QUARTER = B // 4
UNROLL = 8


def _ilv(pos):
  return pos + (pos >> 7) * 896


def _sc_embed(x_cols, tbl_f):
  mesh = plsc.VectorSubcoreMesh(core_axis_name="c", subcore_axis_name="s")

  @functools.partial(
      pl.kernel,
      out_type=jax.ShapeDtypeStruct((OUT_WORDS,), jnp.float32),
      mesh=mesh,
      scratch_types=[
          pltpu.VMEM((8 * HALF,), jnp.float32),
          pltpu.VMEM((HALF,), jnp.float32),
          pltpu.VMEM((HALF,), jnp.float32),
          pltpu.VMEM((2 * EMB_DIM * VOCAB,), jnp.float32),
          pltpu.SemaphoreType.DMA,
          pltpu.SemaphoreType.DMA,
          pltpu.SemaphoreType.DMA,
      ],
      compiler_params=pltpu.CompilerParams(needs_layout_passes=False),
  )
  def k(x_hbm, tbl_hbm, out_hbm, emb_v, ids0_v, ids1_v, tbl_v, sem_in,
        sem_in2, sem_out):
    wid = lax.axis_index("s") * 2 + lax.axis_index("c")
    role = wid // 2
    h = wid % 2

    @pl.when(wid < NUM_CAT)
    def _field_pair():
      f0 = role * 2
      quarter = HALF // 2
      first, second = [], []
      first.append(pltpu.async_copy(
          tbl_hbm.at[pl.ds(f0 * (EMB_DIM * VOCAB), 2 * EMB_DIM * VOCAB)],
          tbl_v, sem_in))
      for c, ids_v in ((f0, ids0_v), (f0 + 1, ids1_v)):
        first.append(pltpu.async_copy(
            x_hbm.at[NUM_NUMERIC + c, pl.ds(h * HALF, quarter)],
            ids_v.at[pl.ds(0, quarter)], sem_in))
        second.append(pltpu.async_copy(
            x_hbm.at[NUM_NUMERIC + c, pl.ds(h * HALF + quarter, quarter)],
            ids_v.at[pl.ds(quarter, quarter)], sem_in2))
      out_base = (role * 2 + h) * 8 * HALF
      outs = []
      for q in range(2):
        for d_i in (first if q == 0 else second):
          d_i.wait()

        @plsc.parallel_loop(q * quarter, (q + 1) * quarter, step=16,
                            unroll=UNROLL)
        def _body(pos):
          lb = _ilv(pos)
          ids0 = ids0_v[pl.ds(pos, 16)].astype(jnp.int32)
          for d in range(EMB_DIM):
            v = plsc.load_gather(tbl_v, [ids0 + d * VOCAB])
            emb_v[pl.ds(lb + d * 128, 16)] = v
          ids1 = ids1_v[pl.ds(pos, 16)].astype(jnp.int32)
          for d in range(EMB_DIM):
            v = plsc.load_gather(tbl_v, [ids1 + (EMB_DIM + d) * VOCAB])
            emb_v[pl.ds(lb + (EMB_DIM + d) * 128, 16)] = v

        outs.append(pltpu.async_copy(
            emb_v.at[pl.ds(q * 4 * HALF, 4 * HALF)],
            out_hbm.at[pl.ds(out_base + q * 4 * HALF, 4 * HALF)], sem_out))
      for d_o in outs:
        d_o.wait()

    def _numeric(base_col, ncols, rbase, nrows, out_off):
      bufs = (ids0_v, ids1_v)
      sems = (sem_in, sem_in2)
      pend = pltpu.async_copy(
          x_hbm.at[base_col, pl.ds(rbase, nrows)],
          bufs[0].at[pl.ds(0, nrows)], sems[0])
      for j in range(ncols):
        nxt = None
        if j + 1 < ncols:
          nxt = pltpu.async_copy(
              x_hbm.at[base_col + j + 1, pl.ds(rbase, nrows)],
              bufs[(j + 1) % 2].at[pl.ds(0, nrows)], sems[(j + 1) % 2])
        pend.wait()
        pend = nxt
        buf = bufs[j % 2]

        @plsc.parallel_loop(0, nrows, step=16, unroll=UNROLL)
        def _copy(pos, j=j, buf=buf):
          emb_v[pl.ds(_ilv(pos) + j * 128, 16)] = buf[pl.ds(pos, 16)]

      pltpu.sync_copy(emb_v.at[pl.ds(0, 8 * nrows)],
                      out_hbm.at[pl.ds(out_off, 8 * nrows)])

    for q in range(4):

      @pl.when(wid == NUM_CAT + q)
      def _numeric_a(q=q):
        _numeric(0, 8, q * QUARTER, QUARTER,
                 13 * (8 * B) + q * 8 * QUARTER)

    for h2 in range(2):

      @pl.when(wid == NUM_CAT + 4 + h2)
      def _numeric_b(h2=h2):
        _numeric(8, 5, h2 * HALF, HALF, 14 * (8 * B) + h2 * 8 * HALF)

  return k(x_cols, tbl_f)


def kernel(x, tables):
  x_cols = x.T
  tbl_f = tables.transpose(0, 2, 1).reshape(-1)
  out_phys = _sc_embed(x_cols, tbl_f)
  out = (out_phys.reshape(ROW_PAD // 8, B // 128, 8, 128)
         .transpose(1, 3, 0, 2)
         .reshape(B, ROW_PAD)[:, :ROW_OUT])
  return out
